# Initial kernel scaffold; baseline (speedup 1.0000x reference)
import jax
import jax.numpy as jnp
from jax.experimental import pallas as pl


def kernel(x, edge_index, batch, tables, W1, b1, W2, b2, W3, b3, lin_w, lin_b):
    raise NotImplementedError("write your pallas kernel here")


# speedup vs baseline: 7.4365x; 7.4365x over previous
import functools

import jax
import jax.numpy as jnp
from jax import lax
from jax.experimental import pallas as pl
from jax.experimental.pallas import tpu as pltpu
from jax.experimental.pallas import tpu_sc as plsc

NN = 10000
EE = 320000
HH = 128
NFEAT = 9
VOC = 64
GG = 256

NC = 2
NS = 16
NW = NC * NS

NP_TILE = 320
N_PAD = NW * NP_TILE
EC = 80
EPW = EC * 128
E_PAD = NW * EPW
GP = 512
GPT = GP // NS

_mesh = plsc.VectorSubcoreMesh(
    core_axis_name="c", subcore_axis_name="s", num_cores=NC, num_subcores=NS)

_Z16 = functools.partial(jnp.zeros, (16,), jnp.float32)


@functools.partial(
    pl.kernel,
    out_type=jax.ShapeDtypeStruct((NW * N_PAD,), jnp.float32),
    mesh=_mesh,
    compiler_params=pltpu.CompilerParams(needs_layout_passes=False),
    scratch_types=[
        pltpu.VMEM((EPW,), jnp.int32),
        pltpu.VMEM((N_PAD,), jnp.float32),
        pltpu.SemaphoreType.DMA,
    ],
)
def _prep(dstf_hbm, degh_hbm, dstv, histv, sem):
    cid = lax.axis_index("c")
    sid = lax.axis_index("s")
    wid = sid * NC + cid

    pltpu.sync_copy(dstf_hbm.at[pl.ds(wid * EPW, EPW)], dstv)

    def zh(i, carry):
        histv[pl.ds(i * 16, 16)] = _Z16()
        return carry
    lax.fori_loop(0, N_PAD // 16, zh, 0)
    ones16 = jnp.ones((16,), jnp.float32)

    def dh(i, carry):
        idx = dstv[pl.ds(i * 16, 16)]
        plsc.addupdate_scatter(histv, [idx], ones16)
        return carry
    lax.fori_loop(0, EPW // 16, dh, 0)
    pltpu.sync_copy(histv, degh_hbm.at[pl.ds(wid * N_PAD, N_PAD)])


@functools.partial(
    pl.kernel,
    out_type=jax.ShapeDtypeStruct((NC, N_PAD, HH), jnp.float32),
    mesh=_mesh,
    compiler_params=pltpu.CompilerParams(needs_layout_passes=False),
    scratch_types=[
        pltpu.VMEM((EC, 128), jnp.int32),
        pltpu.VMEM((16, 128), jnp.int32),
        pltpu.VMEM((128, HH), jnp.float32),
        pltpu.VMEM((128, HH), jnp.float32),
        pltpu.VMEM_SHARED((N_PAD, HH), jnp.float32),
        pltpu.SemaphoreType.DMA,
        pltpu.SemaphoreType.DMA,
        pltpu.SemaphoreType.DMA,
        pltpu.SemaphoreType.DMA,
    ],
)
def _mp(hs_hbm, srcr_hbm, dstr_hbm, zeros_hbm, out_hbm,
        srcv, dstv, bufa, bufb, agg_s, gsa, gsb, ssa, ssb):
    cid = lax.axis_index("c")
    sid = lax.axis_index("s")
    wid = sid * NC + cid

    pltpu.sync_copy(srcr_hbm.at[wid], srcv)

    pltpu.sync_copy(zeros_hbm, bufa)
    nrows = N_PAD // NS
    for z in range(nrows // 128):
        rows = pl.ds(sid * nrows + z * 128, 128)
        pltpu.sync_copy(bufa, agg_s.at[rows])
    plsc.subcore_barrier()

    bufs = (bufa, bufb)
    gsems = (gsa, gsb)
    ssems = (ssa, ssb)

    def gstart(j, s):
        pltpu.async_copy(hs_hbm.at[srcv.at[j]], bufs[s], gsems[s])

    def gwait(j, s):
        pltpu.make_async_copy(hs_hbm.at[srcv.at[j]], bufs[s], gsems[s]).wait()

    def sstart(row, s):
        pltpu.async_copy(bufs[s], agg_s.at[dstv.at[row]], ssems[s], add=True)

    def swait(s):
        pltpu.make_async_copy(bufs[s], agg_s.at[dstv.at[0]], ssems[s]).wait()

    gstart(0, 0)

    def block(b, carry):
        h8 = lax.rem(b, 2) * 8
        pltpu.sync_copy(dstr_hbm.at[wid, pl.ds(8 * b, 8)],
                        dstv.at[pl.ds(h8, 8)])
        for k in range(8):
            j = 8 * b + k
            s = k % 2
            o = 1 - s
            gwait(j, s)
            sstart(h8 + k, s)
            if k < 7:
                if k == 0:
                    @pl.when(b > 0)
                    def _():
                        swait(o)
                else:
                    swait(o)
                gstart(j + 1, o)
            else:
                @pl.when(b < EC // 8 - 1)
                def _():
                    swait(o)
                    gstart(j + 1, o)
        return carry
    lax.fori_loop(0, EC // 8, block, 0)
    swait(0)
    swait(1)
    plsc.subcore_barrier()

    for z in range(nrows // 128):
        rows = pl.ds(sid * nrows + z * 128, 128)
        pltpu.sync_copy(agg_s.at[rows], bufa)
        pltpu.sync_copy(bufa, out_hbm.at[cid, rows])


@functools.partial(
    pl.kernel,
    out_type=jax.ShapeDtypeStruct((NC, GP, HH), jnp.float32),
    mesh=_mesh,
    compiler_params=pltpu.CompilerParams(needs_layout_passes=False),
    scratch_types=[
        pltpu.VMEM((8, 128), jnp.int32),
        pltpu.VMEM((3 * 128, HH), jnp.float32),
        pltpu.VMEM((GPT, HH), jnp.float32),
        pltpu.VMEM_SHARED((GP, HH), jnp.float32),
        pltpu.SemaphoreType.DMA,
    ],
)
def _pool(h3_hbm, batw_hbm, zeros_hbm, out_hbm, bv, hbuf, zbuf, pool_s, sem):
    cid = lax.axis_index("c")
    sid = lax.axis_index("s")
    wid = sid * NC + cid
    base = wid * NP_TILE

    pltpu.sync_copy(batw_hbm.at[wid], bv)
    pltpu.sync_copy(h3_hbm.at[pl.ds(base, NP_TILE)], hbuf.at[pl.ds(0, NP_TILE)])
    pltpu.sync_copy(zeros_hbm.at[pl.ds(0, 3 * 128 - NP_TILE)],
                    hbuf.at[pl.ds(NP_TILE, 3 * 128 - NP_TILE)])
    pltpu.sync_copy(zeros_hbm.at[pl.ds(0, GPT)], zbuf)
    pltpu.sync_copy(zbuf, pool_s.at[pl.ds(sid * GPT, GPT)])
    plsc.subcore_barrier()

    for c in range(3):
        pltpu.sync_copy(hbuf.at[pl.ds(c * 128, 128)], pool_s.at[bv.at[c]],
                        add=True)
    plsc.subcore_barrier()

    pltpu.sync_copy(pool_s.at[pl.ds(sid * GPT, GPT)], zbuf)
    pltpu.sync_copy(zbuf, out_hbm.at[cid, pl.ds(sid * GPT, GPT)])


BLK = 1024
_NBLK = N_PAD // BLK


def _tc1_body(x_ref, bat_ref, dh_ref, tab_ref, w_ref,
              hs_ref, dinv_ref, cnt_ref):
    i = pl.program_id(0)
    xb = x_ref[...]
    tab = tab_ref[...]
    col = lax.broadcasted_iota(jnp.int32, (BLK, VOC), 1)
    h0 = jnp.zeros((BLK, HH), jnp.float32)
    for f in range(NFEAT):
        xf = lax.slice(xb, (0, f), (BLK, f + 1))
        oh = (xf == col).astype(jnp.float32)
        h0 = h0 + jnp.dot(oh, tab[f * VOC:(f + 1) * VOC],
                          preferred_element_type=jnp.float32)

    ones = jnp.ones((NW, 1), jnp.float32)
    deg = lax.dot_general(dh_ref[...], ones, (((0,), (0,)), ((), ()))) + 1.0
    gi = i * BLK + lax.broadcasted_iota(jnp.int32, (BLK, 1), 0)
    dv = jnp.where(gi < NN, lax.rsqrt(deg), 0.0)
    dinv_ref[...] = dv
    hs_ref[...] = jnp.dot(h0, w_ref[...],
                          preferred_element_type=jnp.float32) * dv

    @pl.when(i == 0)
    def _():
        cnt_ref[...] = jnp.zeros((GG, 1), jnp.float32)
    giota = lax.broadcasted_iota(jnp.int32, (GG, BLK), 0)
    oh = (giota == bat_ref[0]).astype(jnp.float32)
    cnt_ref[...] += jnp.dot(oh, jnp.ones((BLK, 1), jnp.float32),
                            preferred_element_type=jnp.float32)


_tc1 = pl.pallas_call(
    _tc1_body,
    grid=(_NBLK,),
    in_specs=[
        pl.BlockSpec((BLK, 16), lambda i: (i, 0)),
        pl.BlockSpec((1, 1, BLK), lambda i: (i, 0, 0)),
        pl.BlockSpec((NW, BLK), lambda i: (0, i)),
        pl.BlockSpec((NFEAT * VOC, HH), lambda i: (0, 0)),
        pl.BlockSpec((HH, HH), lambda i: (0, 0)),
    ],
    out_specs=[
        pl.BlockSpec((BLK, HH), lambda i: (i, 0)),
        pl.BlockSpec((BLK, 1), lambda i: (i, 0)),
        pl.BlockSpec((GG, 1), lambda i: (0, 0)),
    ],
    out_shape=[
        jax.ShapeDtypeStruct((N_PAD, HH), jnp.float32),
        jax.ShapeDtypeStruct((N_PAD, 1), jnp.float32),
        jax.ShapeDtypeStruct((GG, 1), jnp.float32),
    ],
)


def _tc2_body(p0_ref, p1_ref, hs_ref, dinv_ref, b_ref, w_ref, out_ref):
    dv = dinv_ref[...]
    agg = p0_ref[...] + p1_ref[...] + hs_ref[...]
    h = jnp.maximum(agg * dv + b_ref[...], 0.0)
    out_ref[...] = jnp.dot(h, w_ref[...],
                           preferred_element_type=jnp.float32) * dv


_tc2 = pl.pallas_call(
    _tc2_body,
    grid=(_NBLK,),
    in_specs=[
        pl.BlockSpec((BLK, HH), lambda i: (i, 0)),
        pl.BlockSpec((BLK, HH), lambda i: (i, 0)),
        pl.BlockSpec((BLK, HH), lambda i: (i, 0)),
        pl.BlockSpec((BLK, 1), lambda i: (i, 0)),
        pl.BlockSpec((1, HH), lambda i: (0, 0)),
        pl.BlockSpec((HH, HH), lambda i: (0, 0)),
    ],
    out_specs=pl.BlockSpec((BLK, HH), lambda i: (i, 0)),
    out_shape=jax.ShapeDtypeStruct((N_PAD, HH), jnp.float32),
)


def _tc4_body(p0_ref, p1_ref, hs_ref, dinv_ref, b_ref, out_ref):
    agg = p0_ref[...] + p1_ref[...] + hs_ref[...]
    out_ref[...] = agg * dinv_ref[...] + b_ref[...]


_tc4 = pl.pallas_call(
    _tc4_body,
    grid=(_NBLK,),
    in_specs=[
        pl.BlockSpec((BLK, HH), lambda i: (i, 0)),
        pl.BlockSpec((BLK, HH), lambda i: (i, 0)),
        pl.BlockSpec((BLK, HH), lambda i: (i, 0)),
        pl.BlockSpec((BLK, 1), lambda i: (i, 0)),
        pl.BlockSpec((1, HH), lambda i: (0, 0)),
    ],
    out_specs=pl.BlockSpec((BLK, HH), lambda i: (i, 0)),
    out_shape=jax.ShapeDtypeStruct((N_PAD, HH), jnp.float32),
)


def _tc5_body(q0_ref, q1_ref, cnt_ref, lw_ref, lb_ref, out_ref):
    s = (q0_ref[...] + q1_ref[...])[:GG]
    pooled = s / jnp.maximum(cnt_ref[...], 1.0)
    y = jnp.dot(pooled, lw_ref[...],
                preferred_element_type=jnp.float32) + lb_ref[...]
    out_ref[...] = jax.nn.sigmoid(y)


_tc5 = pl.pallas_call(
    _tc5_body,
    out_shape=jax.ShapeDtypeStruct((GG, 1), jnp.float32),
)


def kernel(x, edge_index, batch, tables, W1, b1, W2, b2, W3, b3, lin_w, lin_b):
    x = x.astype(jnp.int32)
    ei = edge_index.astype(jnp.int32)
    batch = batch.astype(jnp.int32)

    tab = tables.reshape(NFEAT * VOC, HH)
    xp = jnp.pad(x, ((0, N_PAD - NN), (0, 7)))
    epad = jnp.full((E_PAD - EE,), NN, jnp.int32)
    src = jnp.concatenate([ei[0], epad]).reshape(NW, EC, 128)
    dstf = jnp.concatenate([ei[1], epad])
    dst = dstf.reshape(NW, EC, 128)
    bat = jnp.concatenate(
        [batch, jnp.full((N_PAD - NN,), GG, jnp.int32)])
    batw = jnp.concatenate(
        [bat.reshape(NW, NP_TILE),
         jnp.full((NW, 8 * 128 - NP_TILE), GG, jnp.int32)],
        axis=1).reshape(NW, 8, 128)
    zeros = jnp.zeros((128, HH), jnp.float32)

    degh = _prep(dstf)
    hs1, dinv, cnt = _tc1(xp, bat.reshape(_NBLK, 1, BLK),
                          degh.reshape(NW, N_PAD), tab, W1)
    p = _mp(hs1, src, dst, zeros)
    hs2 = _tc2(p[0], p[1], hs1, dinv, b1.reshape(1, HH), W2)
    p = _mp(hs2, src, dst, zeros)
    hs3 = _tc2(p[0], p[1], hs2, dinv, b2.reshape(1, HH), W3)
    p = _mp(hs3, src, dst, zeros)
    h3 = _tc4(p[0], p[1], hs3, dinv, b3.reshape(1, HH))
    q = _pool(h3, batw, zeros)
    return _tc5(q[0], q[1], cnt, lin_w, lin_b.reshape(1, 1))
